# Initial kernel scaffold; baseline (speedup 1.0000x reference)
import jax
import jax.numpy as jnp
from jax.experimental import pallas as pl


def kernel(blue_team_indices, red_team_indices, blue_heuristics, red_heuristics, table, W1, b1, W2, b2, W3, b3):
    raise NotImplementedError("write your pallas kernel here")


# speedup vs baseline: 3.9769x; 3.9769x over previous
import functools

import jax
import jax.numpy as jnp
from jax import lax
from jax.experimental import pallas as pl
from jax.experimental.pallas import tpu as pltpu
from jax.experimental.pallas import tpu_sc as plsc

B = 16384
V = 100000
D = 64
H = 32
NUM_SLOTS = 10
E = D * NUM_SLOTS

NC = 2
NS = 16
NW = NC * NS

TOTAL_ROWS = B * NUM_SLOTS
ROWS_PER_W = TOTAL_ROWS // NW
CH = 128
NCH = ROWS_PER_W // CH

BLK = 512


def _sc_gather(table, idx_rs):
    mesh = plsc.VectorSubcoreMesh(core_axis_name="c", subcore_axis_name="s")

    @functools.partial(
        pl.kernel,
        out_type=jax.ShapeDtypeStruct((TOTAL_ROWS, D), jnp.float32),
        mesh=mesh,
        scratch_types=[
            pltpu.VMEM((NCH, CH), jnp.int32),
            pltpu.VMEM((CH, D), jnp.float32),
            pltpu.VMEM((CH, D), jnp.float32),
            pltpu.SemaphoreType.DMA,
            pltpu.SemaphoreType.DMA,
        ],
        compiler_params=pltpu.CompilerParams(use_tc_tiling_on_sc=False),
    )
    def gather_kernel(table_hbm, idx_hbm, out_hbm, idx_v, rows0, rows1, sem0, sem1):
        wid = lax.axis_index("s") * NC + lax.axis_index("c")
        base = wid * ROWS_PER_W
        pltpu.sync_copy(idx_hbm.at[wid], idx_v)

        rows = (rows0, rows1)
        sems = (sem0, sem1)

        cp0 = pltpu.async_copy(table_hbm.at[idx_v.at[0]], rows0, sem0)

        def body(j, _):
            slot = lax.rem(j, 2)
            nxt = lax.rem(j + 1, 2)

            @pl.when(j + 1 < NCH)
            def _():
                for s in range(2):
                    @pl.when(nxt == s)
                    def _():
                        pltpu.async_copy(
                            table_hbm.at[idx_v.at[j + 1]], rows[s], sems[s]
                        )

            for s in range(2):
                @pl.when(slot == s)
                def _():
                    pltpu.make_async_copy(
                        table_hbm.at[idx_v.at[j]], rows[s], sems[s]
                    ).wait()
                    pltpu.sync_copy(rows[s], out_hbm.at[pl.ds(base + j * CH, CH)])
            return 0

        del cp0
        lax.fori_loop(0, NCH, body, 0, unroll=False)

    return gather_kernel(table, idx_rs)


def _mlp_block(g_ref, h_ref, w1e_ref, w1h_ref, b1_ref, w2_ref, b2_ref,
               w3_ref, b3_ref, o_ref):
    h1 = jnp.dot(g_ref[...], w1e_ref[...], preferred_element_type=jnp.float32)
    h1 += jnp.dot(h_ref[...], w1h_ref[...], preferred_element_type=jnp.float32)
    h1 = jnp.maximum(h1 + b1_ref[...], 0.0)
    h2 = jnp.dot(h1, w2_ref[...], preferred_element_type=jnp.float32)
    h2 = jnp.maximum(h2 + b2_ref[...], 0.0)
    z = jnp.dot(h2, w3_ref[...], preferred_element_type=jnp.float32) + b3_ref[...]
    o_ref[...] = jax.nn.sigmoid(z)


def _tc_mlp(g, heur, W1e, W1h, b1, W2, b2, W3, b3):
    grid = (B // BLK,)
    return pl.pallas_call(
        _mlp_block,
        grid=grid,
        in_specs=[
            pl.BlockSpec((BLK, E), lambda i: (i, 0)),
            pl.BlockSpec((BLK, 2 * H), lambda i: (i, 0)),
            pl.BlockSpec((E, 256), lambda i: (0, 0)),
            pl.BlockSpec((2 * H, 256), lambda i: (0, 0)),
            pl.BlockSpec((1, 256), lambda i: (0, 0)),
            pl.BlockSpec((256, 128), lambda i: (0, 0)),
            pl.BlockSpec((1, 128), lambda i: (0, 0)),
            pl.BlockSpec((128, 1), lambda i: (0, 0)),
            pl.BlockSpec((1, 1), lambda i: (0, 0)),
        ],
        out_specs=pl.BlockSpec((BLK, 1), lambda i: (i, 0)),
        out_shape=jax.ShapeDtypeStruct((B, 1), jnp.float32),
    )(g, heur, W1e, W1h, b1, W2, b2, W3, b3)


@jax.jit
def kernel(blue_team_indices, red_team_indices, blue_heuristics,
           red_heuristics, table, W1, b1, W2, b2, W3, b3):
    idx = jnp.concatenate(
        [blue_team_indices, red_team_indices], axis=1
    ).astype(jnp.int32)
    idx_rs = idx.reshape(NW, NCH, CH)
    rows = _sc_gather(table, idx_rs)
    g = rows.reshape(B, E)
    heur = jnp.concatenate([blue_heuristics, red_heuristics], axis=1)
    out = _tc_mlp(
        g, heur,
        W1[:E], W1[E:],
        b1.reshape(1, 256),
        W2, b2.reshape(1, 128),
        W3, b3.reshape(1, 1),
    )
    return out


# speedup vs baseline: 4.0398x; 1.0158x over previous
import functools

import jax
import jax.numpy as jnp
from jax import lax
from jax.experimental import pallas as pl
from jax.experimental.pallas import tpu as pltpu
from jax.experimental.pallas import tpu_sc as plsc

B = 16384
V = 100000
D = 64
H = 32
NUM_SLOTS = 10
E = D * NUM_SLOTS

NC = 2
NS = 16
NW = NC * NS

TOTAL_ROWS = B * NUM_SLOTS
ROWS_PER_W = TOTAL_ROWS // NW
CH = 128
NCH = ROWS_PER_W // CH

BLK = 512


def _sc_gather(table, idx_rs):
    mesh = plsc.VectorSubcoreMesh(core_axis_name="c", subcore_axis_name="s")

    @functools.partial(
        pl.kernel,
        out_type=jax.ShapeDtypeStruct((TOTAL_ROWS, D), jnp.float32),
        mesh=mesh,
        scratch_types=[
            pltpu.VMEM((NCH, CH), jnp.int32),
            pltpu.VMEM((CH, D), jnp.float32),
            pltpu.VMEM((CH, D), jnp.float32),
            pltpu.SemaphoreType.DMA,
            pltpu.SemaphoreType.DMA,
        ],
        compiler_params=pltpu.CompilerParams(use_tc_tiling_on_sc=False),
    )
    def gather_kernel(table_hbm, idx_hbm, out_hbm, idx_v, rows0, rows1, sem0, sem1):
        wid = lax.axis_index("s") * NC + lax.axis_index("c")
        base = wid * ROWS_PER_W
        pltpu.sync_copy(idx_hbm.at[wid], idx_v)

        rows = (rows0, rows1)
        sems = (sem0, sem1)

        cp0 = pltpu.async_copy(table_hbm.at[idx_v.at[0]], rows0, sem0)

        def body(j, _):
            slot = lax.rem(j, 2)
            nxt = lax.rem(j + 1, 2)

            @pl.when(j + 1 < NCH)
            def _():
                for s in range(2):
                    @pl.when(nxt == s)
                    def _():
                        pltpu.async_copy(
                            table_hbm.at[idx_v.at[j + 1]], rows[s], sems[s]
                        )

            for s in range(2):
                @pl.when(slot == s)
                def _():
                    pltpu.make_async_copy(
                        table_hbm.at[idx_v.at[j]], rows[s], sems[s]
                    ).wait()
                    pltpu.sync_copy(rows[s], out_hbm.at[pl.ds(base + j * CH, CH)])
            return 0

        del cp0
        lax.fori_loop(0, NCH, body, 0, unroll=False)

    return gather_kernel(table, idx_rs)


def _mlp_block(g_ref, h_ref, w1e_ref, w1h_ref, b1_ref, w2_ref, b2_ref,
               w3_ref, b3_ref, o_ref):
    h1 = jnp.dot(h_ref[...], w1h_ref[...], preferred_element_type=jnp.float32)
    for p in range(5):
        h1 += jnp.dot(g_ref[p], w1e_ref[p], preferred_element_type=jnp.float32)
    h1 = jnp.maximum(h1 + b1_ref[...], 0.0)
    h2 = jnp.dot(h1, w2_ref[...], preferred_element_type=jnp.float32)
    h2 = jnp.maximum(h2 + b2_ref[...], 0.0)
    z = jnp.dot(h2, w3_ref[...], preferred_element_type=jnp.float32) + b3_ref[...]
    o_ref[...] = jax.nn.sigmoid(z)


def _tc_mlp(g3, heur, W1e3, W1h, b1, W2, b2, W3, b3):
    grid = (B // BLK,)
    return pl.pallas_call(
        _mlp_block,
        grid=grid,
        in_specs=[
            pl.BlockSpec((5, BLK, 128), lambda i: (0, i, 0)),
            pl.BlockSpec((BLK, 2 * H), lambda i: (i, 0)),
            pl.BlockSpec((5, 128, 256), lambda i: (0, 0, 0)),
            pl.BlockSpec((2 * H, 256), lambda i: (0, 0)),
            pl.BlockSpec((1, 256), lambda i: (0, 0)),
            pl.BlockSpec((256, 128), lambda i: (0, 0)),
            pl.BlockSpec((1, 128), lambda i: (0, 0)),
            pl.BlockSpec((128, 1), lambda i: (0, 0)),
            pl.BlockSpec((1, 1), lambda i: (0, 0)),
        ],
        out_specs=pl.BlockSpec((BLK, 1), lambda i: (i, 0)),
        out_shape=jax.ShapeDtypeStruct((B, 1), jnp.float32),
    )(g3, heur, W1e3, W1h, b1, W2, b2, W3, b3)


@jax.jit
def kernel(blue_team_indices, red_team_indices, blue_heuristics,
           red_heuristics, table, W1, b1, W2, b2, W3, b3):
    idx = jnp.concatenate(
        [blue_team_indices, red_team_indices], axis=1
    ).astype(jnp.int32)
    idx_pm = idx.reshape(B, 5, 2).transpose(1, 0, 2)
    idx_rs = idx_pm.reshape(NW, NCH, CH)
    rows = _sc_gather(table, idx_rs)
    g3 = rows.reshape(5, B, 2 * D)
    heur = jnp.concatenate([blue_heuristics, red_heuristics], axis=1)
    out = _tc_mlp(
        g3, heur,
        W1[:E].reshape(5, 2 * D, 256), W1[E:],
        b1.reshape(1, 256),
        W2, b2.reshape(1, 128),
        W3, b3.reshape(1, 1),
    )
    return out
